# baseline (speedup 1.0000x reference)
import functools

import jax
import jax.numpy as jnp
from jax import lax
from jax.experimental import pallas as pl
from jax.experimental.pallas import tpu as pltpu
from jax.experimental.pallas import tpu_sc as plsc

NC, NS = 2, 16
NW = NC * NS
D = 512
L = 64
C = 32
NB = 3


def _pos_body(out_ref):
    pos = lax.broadcasted_iota(jnp.int32, (L, D), 0).astype(jnp.float32)
    d = lax.broadcasted_iota(jnp.int32, (L, D), 1)
    k2 = ((d // 2) * 2).astype(jnp.float32)
    freq = jnp.exp(k2 * (-jnp.log(10000.0) / D))
    angle = pos * freq
    out_ref[...] = jnp.where(d % 2 == 0, jnp.cos(angle), jnp.sin(angle))


@jax.jit
def _pos_table():
    return pl.pallas_call(
        _pos_body,
        out_shape=jax.ShapeDtypeStruct((L, D), jnp.float32),
    )()


@functools.partial(jax.jit, static_argnames=("n_rows",))
def _sc_gather(idx, packed_table, pos, *, n_rows):
    b_per_w = n_rows // NW
    n_chunks = b_per_w // C
    n_main = (n_chunks // NB) * NB

    def body(idx_hbm, table_hbm, pos_hbm, out_hbm,
             idx_v, g0, g1, g2, o0, o1, o2, pos_v,
             gsem0, gsem1, gsem2, osem0, osem1, osem2):
        gbuf = [g0, g1, g2]
        obuf = [o0, o1, o2]
        gsems = [gsem0, gsem1, gsem2]
        osems = [osem0, osem1, osem2]
        wid = lax.axis_index("s") * NC + lax.axis_index("c")
        base = wid * b_per_w
        pltpu.sync_copy(pos_hbm, pos_v)
        pltpu.sync_copy(idx_hbm.at[pl.ds(base, b_per_w)], idx_v)

        def gather_copy(g, b):
            return pltpu.make_async_copy(
                table_hbm.at[idx_v.at[pl.ds(g * C, C)]], gbuf[b], gsems[b])

        def out_copy(g, b):
            return pltpu.make_async_copy(
                obuf[b], out_hbm.at[pl.ds(base + g * C, C)], osems[b])

        def add_pos(gc, b):
            p0 = (gc % 2) * C

            @plsc.parallel_loop(0, C, 1, unroll=2)
            def row(r):
                for j in range(D // 32):
                    vi = gbuf[b][r, pl.ds(j * 16, 16)]
                    lo = lax.bitcast_convert_type(
                        lax.shift_left(vi, 16), jnp.float32)
                    hi = lax.bitcast_convert_type(
                        lax.bitwise_and(vi, jnp.int32(-65536)), jnp.float32)
                    s0 = pl.ds(j * 32, 16)
                    s1 = pl.ds(j * 32 + 16, 16)
                    obuf[b][r, s0] = lo + pos_v[p0 + r, s0]
                    obuf[b][r, s1] = hi + pos_v[p0 + r, s1]

        def chunk_step(gc, b):
            b2 = (b + 2) % NB
            gather_copy(gc, b).wait()

            @pl.when(gc + 2 < n_chunks)
            def _():
                gather_copy(gc + 2, b2).start()

            @pl.when(gc >= NB)
            def _():
                out_copy(gc - NB, b).wait()

            add_pos(gc, b)
            add_pos(gc, b)
            out_copy(gc, b).start()

        gather_copy(0, 0).start()
        gather_copy(1, 1).start()

        def step(t, carry):
            g = t * NB
            for b in range(NB):
                chunk_step(g + b, b)
            return carry

        lax.fori_loop(0, n_main // NB, step, 0)
        for gc in range(n_main, n_chunks):
            chunk_step(gc, gc % NB)
        for gc in range(n_chunks - NB, n_chunks):
            out_copy(gc, gc % NB).wait()

    return pl.kernel(
        body,
        out_type=jax.ShapeDtypeStruct((n_rows, D), jnp.float32),
        mesh=plsc.VectorSubcoreMesh(core_axis_name="c", subcore_axis_name="s"),
        scratch_types=[
            pltpu.VMEM((b_per_w,), jnp.int32),
            pltpu.VMEM((C, D // 2), jnp.int32),
            pltpu.VMEM((C, D // 2), jnp.int32),
            pltpu.VMEM((C, D // 2), jnp.int32),
            pltpu.VMEM((C, D), jnp.float32),
            pltpu.VMEM((C, D), jnp.float32),
            pltpu.VMEM((C, D), jnp.float32),
            pltpu.VMEM((L, D), jnp.float32),
            pltpu.SemaphoreType.DMA,
            pltpu.SemaphoreType.DMA,
            pltpu.SemaphoreType.DMA,
            pltpu.SemaphoreType.DMA,
            pltpu.SemaphoreType.DMA,
            pltpu.SemaphoreType.DMA,
        ],
    )(idx, packed_table, pos)


def kernel(inputs, table):
    batch, seq = inputs.shape
    vocab = table.shape[0]
    idx = inputs.reshape(-1).astype(jnp.int32)
    tb = table.astype(jnp.bfloat16)
    ti = tb.reshape(vocab, 16, 2, 16).transpose(0, 1, 3, 2).reshape(vocab, D)
    ti32 = lax.bitcast_convert_type(ti.reshape(vocab, D // 2, 2), jnp.int32)
    pos = _pos_table()
    out = _sc_gather(idx, ti32, pos, n_rows=batch * seq)
    return out.reshape(batch, seq, D)


# speedup vs baseline: 3.0600x; 3.0600x over previous
import functools

import jax
import jax.numpy as jnp
from jax import lax
from jax.experimental import pallas as pl
from jax.experimental.pallas import tpu as pltpu
from jax.experimental.pallas import tpu_sc as plsc

NC, NS = 2, 16
NW = NC * NS
D = 512
L = 64
C = 32
NB = 3


def _pos_body(out_ref):
    pos = lax.broadcasted_iota(jnp.int32, (L, D), 0).astype(jnp.float32)
    d = lax.broadcasted_iota(jnp.int32, (L, D), 1)
    k2 = ((d // 2) * 2).astype(jnp.float32)
    freq = jnp.exp(k2 * (-jnp.log(10000.0) / D))
    angle = pos * freq
    out_ref[...] = jnp.where(d % 2 == 0, jnp.cos(angle), jnp.sin(angle))


@jax.jit
def _pos_table():
    return pl.pallas_call(
        _pos_body,
        out_shape=jax.ShapeDtypeStruct((L, D), jnp.float32),
    )()


@functools.partial(jax.jit, static_argnames=("n_rows",))
def _sc_gather(idx, packed_table, pos, *, n_rows):
    b_per_w = n_rows // NW
    n_chunks = b_per_w // C
    n_main = (n_chunks // NB) * NB

    def body(idx_hbm, table_hbm, pos_hbm, out_hbm,
             idx_v, g0, g1, g2, o0, o1, o2, pos_v,
             gsem0, gsem1, gsem2, osem0, osem1, osem2):
        gbuf = [g0, g1, g2]
        obuf = [o0, o1, o2]
        gsems = [gsem0, gsem1, gsem2]
        osems = [osem0, osem1, osem2]
        wid = lax.axis_index("s") * NC + lax.axis_index("c")
        base = wid * b_per_w
        pltpu.sync_copy(pos_hbm, pos_v)
        pltpu.sync_copy(idx_hbm.at[pl.ds(base, b_per_w)], idx_v)

        def gather_copy(g, b):
            return pltpu.make_async_copy(
                table_hbm.at[idx_v.at[pl.ds(g * C, C)]], gbuf[b], gsems[b])

        def out_copy(g, b):
            return pltpu.make_async_copy(
                obuf[b], out_hbm.at[pl.ds(base + g * C, C)], osems[b])

        def add_pos(gc, b):
            p0 = (gc % 2) * C

            @plsc.parallel_loop(0, C, 1, unroll=2)
            def row(r):
                for j in range(D // 32):
                    sp = pl.ds(j * 16, 16)
                    vi = gbuf[b][r, sp]
                    pi = pos_v[p0 + r, sp]
                    lo = lax.bitcast_convert_type(
                        lax.shift_left(vi, 16), jnp.float32)
                    hi = lax.bitcast_convert_type(
                        lax.bitwise_and(vi, jnp.int32(-65536)), jnp.float32)
                    plo = lax.bitcast_convert_type(
                        lax.shift_left(pi, 16), jnp.float32)
                    phi = lax.bitcast_convert_type(
                        lax.bitwise_and(pi, jnp.int32(-65536)), jnp.float32)
                    obuf[b][r, pl.ds(j * 32, 16)] = lo + plo
                    obuf[b][r, pl.ds(j * 32 + 16, 16)] = hi + phi

        def chunk_step(gc, b):
            b2 = (b + 2) % NB
            gather_copy(gc, b).wait()

            @pl.when(gc + 2 < n_chunks)
            def _():
                gather_copy(gc + 2, b2).start()

            @pl.when(gc >= NB)
            def _():
                out_copy(gc - NB, b).wait()

            add_pos(gc, b)
            out_copy(gc, b).start()

        gather_copy(0, 0).start()
        gather_copy(1, 1).start()

        def step(t, carry):
            g = t * NB
            for b in range(NB):
                chunk_step(g + b, b)
            return carry

        lax.fori_loop(0, n_main // NB, step, 0)
        for gc in range(n_main, n_chunks):
            chunk_step(gc, gc % NB)
        for gc in range(n_chunks - NB, n_chunks):
            out_copy(gc, gc % NB).wait()

    return pl.kernel(
        body,
        out_type=jax.ShapeDtypeStruct((n_rows, D), jnp.float32),
        mesh=plsc.VectorSubcoreMesh(core_axis_name="c", subcore_axis_name="s"),
        scratch_types=[
            pltpu.VMEM((b_per_w,), jnp.int32),
            pltpu.VMEM((C, D // 2), jnp.int32),
            pltpu.VMEM((C, D // 2), jnp.int32),
            pltpu.VMEM((C, D // 2), jnp.int32),
            pltpu.VMEM((C, D), jnp.float32),
            pltpu.VMEM((C, D), jnp.float32),
            pltpu.VMEM((C, D), jnp.float32),
            pltpu.VMEM((L, D // 2), jnp.int32),
            pltpu.SemaphoreType.DMA,
            pltpu.SemaphoreType.DMA,
            pltpu.SemaphoreType.DMA,
            pltpu.SemaphoreType.DMA,
            pltpu.SemaphoreType.DMA,
            pltpu.SemaphoreType.DMA,
        ],
    )(idx, packed_table, pos)


def kernel(inputs, table):
    batch, seq = inputs.shape
    vocab = table.shape[0]
    idx = inputs.reshape(-1).astype(jnp.int32)
    tb = table.astype(jnp.bfloat16)
    ti = tb.reshape(vocab, 16, 2, 16).transpose(0, 1, 3, 2).reshape(vocab, D)
    ti32 = lax.bitcast_convert_type(ti.reshape(vocab, D // 2, 2), jnp.int32)
    pos = _pos_table()
    pb = pos.astype(jnp.bfloat16)
    pi = pb.reshape(L, 16, 2, 16).transpose(0, 1, 3, 2).reshape(L, D)
    pi32 = lax.bitcast_convert_type(pi.reshape(L, D // 2, 2), jnp.int32)
    out = _sc_gather(idx, ti32, pi32, n_rows=batch * seq)
    return out.reshape(batch, seq, D)


# speedup vs baseline: 3.0632x; 1.0010x over previous
import functools

import jax
import jax.numpy as jnp
from jax import lax
from jax.experimental import pallas as pl
from jax.experimental.pallas import tpu as pltpu
from jax.experimental.pallas import tpu_sc as plsc

NC, NS = 2, 16
NW = NC * NS
D = 512
L = 64
C = 32
NB = 4


def _pos_body(out_ref):
    pos = lax.broadcasted_iota(jnp.int32, (L, D), 0).astype(jnp.float32)
    d = lax.broadcasted_iota(jnp.int32, (L, D), 1)
    k2 = ((d // 2) * 2).astype(jnp.float32)
    freq = jnp.exp(k2 * (-jnp.log(10000.0) / D))
    angle = pos * freq
    out_ref[...] = jnp.where(d % 2 == 0, jnp.cos(angle), jnp.sin(angle))


@jax.jit
def _pos_table():
    return pl.pallas_call(
        _pos_body,
        out_shape=jax.ShapeDtypeStruct((L, D), jnp.float32),
    )()


@functools.partial(jax.jit, static_argnames=("n_rows",))
def _sc_gather(idx, packed_table, pos, *, n_rows):
    b_per_w = n_rows // NW
    n_chunks = b_per_w // C
    n_main = (n_chunks // NB) * NB

    def body(idx_hbm, table_hbm, pos_hbm, out_hbm,
             idx_v, g0, g1, g2, g3, o0, o1, o2, o3, pos_v,
             gsem0, gsem1, gsem2, gsem3, osem0, osem1, osem2, osem3):
        gbuf = [g0, g1, g2, g3]
        obuf = [o0, o1, o2, o3]
        gsems = [gsem0, gsem1, gsem2, gsem3]
        osems = [osem0, osem1, osem2, osem3]
        wid = lax.axis_index("s") * NC + lax.axis_index("c")
        base = wid * b_per_w
        pltpu.sync_copy(pos_hbm, pos_v)
        pltpu.sync_copy(idx_hbm.at[pl.ds(base, b_per_w)], idx_v)

        def gather_copy(g, b):
            return pltpu.make_async_copy(
                table_hbm.at[idx_v.at[pl.ds(g * C, C)]], gbuf[b], gsems[b])

        def out_copy(g, b):
            return pltpu.make_async_copy(
                obuf[b], out_hbm.at[pl.ds(base + g * C, C)], osems[b])

        def add_pos(gc, b):
            p0 = (gc % 2) * C

            @plsc.parallel_loop(0, C, 1, unroll=2)
            def row(r):
                for j in range(D // 32):
                    sp = pl.ds(j * 16, 16)
                    vi = gbuf[b][r, sp]
                    pi = pos_v[p0 + r, sp]
                    lo = lax.bitcast_convert_type(
                        lax.shift_left(vi, 16), jnp.float32)
                    hi = lax.bitcast_convert_type(
                        lax.bitwise_and(vi, jnp.int32(-65536)), jnp.float32)
                    plo = lax.bitcast_convert_type(
                        lax.shift_left(pi, 16), jnp.float32)
                    phi = lax.bitcast_convert_type(
                        lax.bitwise_and(pi, jnp.int32(-65536)), jnp.float32)
                    obuf[b][r, pl.ds(j * 32, 16)] = lo + plo
                    obuf[b][r, pl.ds(j * 32 + 16, 16)] = hi + phi

        def chunk_step(gc, b):
            b3 = (b + 3) % NB
            gather_copy(gc, b).wait()

            @pl.when(gc + 3 < n_chunks)
            def _():
                gather_copy(gc + 3, b3).start()

            @pl.when(gc >= NB)
            def _():
                out_copy(gc - NB, b).wait()

            add_pos(gc, b)
            out_copy(gc, b).start()

        gather_copy(0, 0).start()
        gather_copy(1, 1).start()
        gather_copy(2, 2).start()

        def step(t, carry):
            g = t * NB
            for b in range(NB):
                chunk_step(g + b, b)
            return carry

        lax.fori_loop(0, n_main // NB, step, 0)
        for gc in range(n_main, n_chunks):
            chunk_step(gc, gc % NB)
        for gc in range(n_chunks - NB, n_chunks):
            out_copy(gc, gc % NB).wait()

    return pl.kernel(
        body,
        out_type=jax.ShapeDtypeStruct((n_rows, D), jnp.float32),
        mesh=plsc.VectorSubcoreMesh(core_axis_name="c", subcore_axis_name="s"),
        scratch_types=[
            pltpu.VMEM((b_per_w,), jnp.int32),
            pltpu.VMEM((C, D // 2), jnp.int32),
            pltpu.VMEM((C, D // 2), jnp.int32),
            pltpu.VMEM((C, D // 2), jnp.int32),
            pltpu.VMEM((C, D // 2), jnp.int32),
            pltpu.VMEM((C, D), jnp.float32),
            pltpu.VMEM((C, D), jnp.float32),
            pltpu.VMEM((C, D), jnp.float32),
            pltpu.VMEM((C, D), jnp.float32),
            pltpu.VMEM((L, D // 2), jnp.int32),
            pltpu.SemaphoreType.DMA,
            pltpu.SemaphoreType.DMA,
            pltpu.SemaphoreType.DMA,
            pltpu.SemaphoreType.DMA,
            pltpu.SemaphoreType.DMA,
            pltpu.SemaphoreType.DMA,
            pltpu.SemaphoreType.DMA,
            pltpu.SemaphoreType.DMA,
        ],
    )(idx, packed_table, pos)


def kernel(inputs, table):
    batch, seq = inputs.shape
    vocab = table.shape[0]
    idx = inputs.reshape(-1).astype(jnp.int32)
    tb = table.astype(jnp.bfloat16)
    ti = tb.reshape(vocab, 16, 2, 16).transpose(0, 1, 3, 2).reshape(vocab, D)
    ti32 = lax.bitcast_convert_type(ti.reshape(vocab, D // 2, 2), jnp.int32)
    pos = _pos_table()
    pb = pos.astype(jnp.bfloat16)
    pi = pb.reshape(L, 16, 2, 16).transpose(0, 1, 3, 2).reshape(L, D)
    pi32 = lax.bitcast_convert_type(pi.reshape(L, D // 2, 2), jnp.int32)
    out = _sc_gather(idx, ti32, pi32, n_rows=batch * seq)
    return out.reshape(batch, seq, D)
